# baseline (speedup 1.0000x reference)
import functools

import jax
import jax.numpy as jnp
from jax import lax
from jax.experimental import pallas as pl
from jax.experimental.pallas import tpu as pltpu
from jax.experimental.pallas import tpu_sc as plsc

EMBED = 128
N_PLAYERS = 6
N_TYPES = 3
BATCH = 16384
NC, NS = 2, 16
NW = NC * NS
B_PER_W = BATCH // NW
NB = 16
NCH = B_PER_W // NB


@functools.cache
def _build_sc_add():
  mesh = plsc.VectorSubcoreMesh(core_axis_name="c", subcore_axis_name="s")

  @functools.partial(
      pl.kernel,
      out_type=jax.ShapeDtypeStruct((BATCH, N_PLAYERS, EMBED), jnp.float32),
      mesh=mesh,
      compiler_params=pltpu.CompilerParams(needs_layout_passes=False),
      scratch_types=[
          pltpu.VMEM((NB, N_PLAYERS, EMBED), jnp.float32),
          pltpu.VMEM((NB, N_PLAYERS, EMBED), jnp.float32),
          pltpu.VMEM((NB, N_PLAYERS, EMBED), jnp.float32),
          pltpu.VMEM((NB, N_PLAYERS, EMBED), jnp.float32),
          pltpu.VMEM((NB * N_PLAYERS + 16,), jnp.int32),
          pltpu.VMEM((NB * N_PLAYERS + 16,), jnp.int32),
          pltpu.VMEM((N_PLAYERS * EMBED,), jnp.float32),
          pltpu.VMEM((N_TYPES * EMBED,), jnp.float32),
          pltpu.VMEM((N_PLAYERS * N_TYPES * EMBED,), jnp.float32),
          pltpu.SemaphoreType.DMA((2,)),
          pltpu.SemaphoreType.DMA((2,)),
          pltpu.SemaphoreType.DMA((2,)),
      ],
  )
  def _sc_add(x_hbm, t_hbm, te_hbm, pe_hbm, out_hbm,
              xbuf0, xbuf1, obuf0, obuf1, tbuf0, tbuf1,
              pe_v, te_v, cbuf,
              xin_sem, tin_sem, out_sem):
    xbufs = (xbuf0, xbuf1)
    obufs = (obuf0, obuf1)
    tbufs = (tbuf0, tbuf1)
    wid = lax.axis_index("s") * NC + lax.axis_index("c")
    base = wid * B_PER_W

    pltpu.sync_copy(pe_hbm, pe_v)
    pltpu.sync_copy(te_hbm, te_v)
    for p in range(N_PLAYERS):
      for t in range(N_TYPES):
        for j in range(EMBED // 16):
          cbuf[pl.ds((p * N_TYPES + t) * EMBED + j * 16, 16)] = (
              pe_v[pl.ds(p * EMBED + j * 16, 16)]
              + te_v[pl.ds(t * EMBED + j * 16, 16)])

    def start_in(g, b):
      pltpu.async_copy(x_hbm.at[pl.ds(base + g * NB, NB)],
                       xbufs[b], xin_sem.at[b])
      pltpu.async_copy(t_hbm.at[pl.ds((base + g * NB) * N_PLAYERS,
                                      NB * N_PLAYERS)],
                       tbufs[b].at[pl.ds(0, NB * N_PLAYERS)], tin_sem.at[b])


    def wait_in(b):
      pltpu.make_async_copy(x_hbm.at[pl.ds(0, NB)], xbufs[b],
                            xin_sem.at[b]).wait()
      pltpu.make_async_copy(t_hbm.at[pl.ds(0, NB * N_PLAYERS)],
                            tbufs[b].at[pl.ds(0, NB * N_PLAYERS)],
                            tin_sem.at[b]).wait()

    def start_out(g, b):
      pltpu.async_copy(obufs[b],
                       out_hbm.at[pl.ds(base + g * NB, NB)],
                       out_sem.at[b])

    def wait_out(b):
      pltpu.make_async_copy(obufs[b], out_hbm.at[pl.ds(0, NB)],
                            out_sem.at[b]).wait()

    def compute(b):
      @plsc.parallel_loop(0, NB, unroll=4)
      def batch_body(bi):
        tv = tbufs[b][pl.ds(bi * N_PLAYERS, 16)]
        for p in range(N_PLAYERS):
          t = tv[p]
          cib = (p * N_TYPES + t) * EMBED
          xr = xbufs[b].at[bi, p]
          orr = obufs[b].at[bi, p]
          for jb in range(EMBED // 16):
            sl = pl.ds(jb * 16, 16)
            orr[sl] = xr[sl] + cbuf[pl.ds(cib + jb * 16, 16)]

    start_in(0, 0)
    start_in(1, 1)

    def pair_body(gg, _):
      for b in range(2):
        g = 2 * gg + b
        wait_in(b)

        @pl.when(gg >= 1)
        def _():
          wait_out(b)

        compute(b)
        start_out(g, b)

        @pl.when(g + 2 < NCH)
        def _():
          start_in(g + 2, b)
      return 0

    lax.fori_loop(0, NCH // 2, pair_body, 0)
    wait_out(0)
    wait_out(1)

  return _sc_add


def kernel(x, entity_types, entity_type_embedding, position_embedding):
  t_flat = entity_types.reshape(-1).astype(jnp.int32)
  out = _build_sc_add()(x, t_flat,
                        entity_type_embedding.reshape(-1),
                        position_embedding.reshape(-1))
  return out


# speedup vs baseline: 1.0110x; 1.0110x over previous
import functools

import jax
import jax.numpy as jnp
from jax import lax
from jax.experimental import pallas as pl
from jax.experimental.pallas import tpu as pltpu
from jax.experimental.pallas import tpu_sc as plsc

EMBED = 128
N_PLAYERS = 6
N_TYPES = 3
BATCH = 16384
NC, NS = 2, 16
NW = NC * NS
B_PER_W = BATCH // NW
NB = 16
NCH = B_PER_W // NB


@functools.cache
def _build_sc_add():
  mesh = plsc.VectorSubcoreMesh(core_axis_name="c", subcore_axis_name="s")

  @functools.partial(
      pl.kernel,
      out_type=jax.ShapeDtypeStruct((BATCH, N_PLAYERS, EMBED), jnp.float32),
      mesh=mesh,
      compiler_params=pltpu.CompilerParams(
          needs_layout_passes=False,
          skip_device_barrier=True,
          disable_bounds_checks=True,
      ),
      scratch_types=[
          pltpu.VMEM((NB, N_PLAYERS, EMBED), jnp.float32),
          pltpu.VMEM((NB, N_PLAYERS, EMBED), jnp.float32),
          pltpu.VMEM((NB, N_PLAYERS, EMBED), jnp.float32),
          pltpu.VMEM((NB, N_PLAYERS, EMBED), jnp.float32),
          pltpu.VMEM((NB * N_PLAYERS + 16,), jnp.int32),
          pltpu.VMEM((NB * N_PLAYERS + 16,), jnp.int32),
          pltpu.VMEM((N_PLAYERS * EMBED,), jnp.float32),
          pltpu.VMEM((N_TYPES * EMBED,), jnp.float32),
          pltpu.VMEM((N_PLAYERS * N_TYPES * EMBED,), jnp.float32),
          pltpu.SemaphoreType.DMA((2,)),
          pltpu.SemaphoreType.DMA((2,)),
          pltpu.SemaphoreType.DMA((2,)),
      ],
  )
  def _sc_add(x_hbm, t_hbm, te_hbm, pe_hbm, out_hbm,
              xbuf0, xbuf1, obuf0, obuf1, tbuf0, tbuf1,
              pe_v, te_v, cbuf,
              xin_sem, tin_sem, out_sem):
    xbufs = (xbuf0, xbuf1)
    obufs = (obuf0, obuf1)
    tbufs = (tbuf0, tbuf1)
    wid = lax.axis_index("s") * NC + lax.axis_index("c")
    base = wid * B_PER_W

    pltpu.sync_copy(pe_hbm, pe_v)
    pltpu.sync_copy(te_hbm, te_v)
    for p in range(N_PLAYERS):
      for t in range(N_TYPES):
        for j in range(EMBED // 16):
          cbuf[pl.ds((p * N_TYPES + t) * EMBED + j * 16, 16)] = (
              pe_v[pl.ds(p * EMBED + j * 16, 16)]
              + te_v[pl.ds(t * EMBED + j * 16, 16)])

    def start_in(g, b):
      pltpu.async_copy(x_hbm.at[pl.ds(base + g * NB, NB)],
                       xbufs[b], xin_sem.at[b])
      pltpu.async_copy(t_hbm.at[pl.ds((base + g * NB) * N_PLAYERS,
                                      NB * N_PLAYERS)],
                       tbufs[b].at[pl.ds(0, NB * N_PLAYERS)], tin_sem.at[b])


    def wait_in(b):
      pltpu.make_async_copy(x_hbm.at[pl.ds(0, NB)], xbufs[b],
                            xin_sem.at[b]).wait()
      pltpu.make_async_copy(t_hbm.at[pl.ds(0, NB * N_PLAYERS)],
                            tbufs[b].at[pl.ds(0, NB * N_PLAYERS)],
                            tin_sem.at[b]).wait()

    def start_out(g, b):
      pltpu.async_copy(obufs[b],
                       out_hbm.at[pl.ds(base + g * NB, NB)],
                       out_sem.at[b])

    def wait_out(b):
      pltpu.make_async_copy(obufs[b], out_hbm.at[pl.ds(0, NB)],
                            out_sem.at[b]).wait()

    def compute(b):
      @plsc.parallel_loop(0, NB, unroll=2)
      def batch_body(bi):
        tv = tbufs[b][pl.ds(bi * N_PLAYERS, 16)]
        for p in range(N_PLAYERS):
          t = tv[p]
          cib = (p * N_TYPES + t) * EMBED
          xr = xbufs[b].at[bi, p]
          orr = obufs[b].at[bi, p]
          for jb in range(EMBED // 16):
            sl = pl.ds(jb * 16, 16)
            orr[sl] = xr[sl] + cbuf[pl.ds(cib + jb * 16, 16)]

    start_in(0, 0)
    start_in(1, 1)

    def pair_body(gg, _):
      for b in range(2):
        g = 2 * gg + b
        wait_in(b)

        @pl.when(gg >= 1)
        def _():
          wait_out(b)

        compute(b)
        start_out(g, b)

        @pl.when(g + 2 < NCH)
        def _():
          start_in(g + 2, b)
      return 0

    lax.fori_loop(0, NCH // 2, pair_body, 0)
    wait_out(0)
    wait_out(1)

  return _sc_add


def kernel(x, entity_types, entity_type_embedding, position_embedding):
  t_flat = entity_types.reshape(-1).astype(jnp.int32)
  out = _build_sc_add()(x, t_flat,
                        entity_type_embedding.reshape(-1),
                        position_embedding.reshape(-1))
  return out


# speedup vs baseline: 1.0203x; 1.0092x over previous
import functools

import jax
import jax.numpy as jnp
from jax import lax
from jax.experimental import pallas as pl
from jax.experimental.pallas import tpu as pltpu
from jax.experimental.pallas import tpu_sc as plsc

EMBED = 128
N_PLAYERS = 6
N_TYPES = 3
BATCH = 16384
NC, NS = 2, 16
NW = NC * NS
B_PER_W = BATCH // NW
NB = 16
NCH = B_PER_W // NB


@functools.cache
def _build_sc_add():
  mesh = plsc.VectorSubcoreMesh(core_axis_name="c", subcore_axis_name="s")

  @functools.partial(
      pl.kernel,
      out_type=jax.ShapeDtypeStruct((BATCH, N_PLAYERS, EMBED), jnp.float32),
      mesh=mesh,
      compiler_params=pltpu.CompilerParams(
          needs_layout_passes=False,
          skip_device_barrier=True,
          disable_bounds_checks=True,
          use_tc_tiling_on_sc=True,
      ),
      scratch_types=[
          pltpu.VMEM((NB, N_PLAYERS, EMBED), jnp.float32),
          pltpu.VMEM((NB, N_PLAYERS, EMBED), jnp.float32),
          pltpu.VMEM((NB, N_PLAYERS, EMBED), jnp.float32),
          pltpu.VMEM((NB, N_PLAYERS, EMBED), jnp.float32),
          pltpu.VMEM((NB * N_PLAYERS + 16,), jnp.int32),
          pltpu.VMEM((NB * N_PLAYERS + 16,), jnp.int32),
          pltpu.VMEM((N_PLAYERS * EMBED,), jnp.float32),
          pltpu.VMEM((N_TYPES * EMBED,), jnp.float32),
          pltpu.VMEM((N_PLAYERS * N_TYPES * EMBED,), jnp.float32),
          pltpu.SemaphoreType.DMA((2,)),
          pltpu.SemaphoreType.DMA((2,)),
          pltpu.SemaphoreType.DMA((2,)),
      ],
  )
  def _sc_add(x_hbm, t_hbm, te_hbm, pe_hbm, out_hbm,
              xbuf0, xbuf1, obuf0, obuf1, tbuf0, tbuf1,
              pe_v, te_v, cbuf,
              xin_sem, tin_sem, out_sem):
    xbufs = (xbuf0, xbuf1)
    obufs = (obuf0, obuf1)
    tbufs = (tbuf0, tbuf1)
    wid = lax.axis_index("s") * NC + lax.axis_index("c")
    base = wid * B_PER_W

    pltpu.sync_copy(pe_hbm, pe_v)
    pltpu.sync_copy(te_hbm, te_v)
    for p in range(N_PLAYERS):
      for t in range(N_TYPES):
        for j in range(EMBED // 16):
          cbuf[pl.ds((p * N_TYPES + t) * EMBED + j * 16, 16)] = (
              pe_v[pl.ds(p * EMBED + j * 16, 16)]
              + te_v[pl.ds(t * EMBED + j * 16, 16)])

    def start_in(g, b):
      pltpu.async_copy(x_hbm.at[pl.ds(base + g * NB, NB)],
                       xbufs[b], xin_sem.at[b])
      pltpu.async_copy(t_hbm.at[pl.ds((base + g * NB) * N_PLAYERS,
                                      NB * N_PLAYERS)],
                       tbufs[b].at[pl.ds(0, NB * N_PLAYERS)], tin_sem.at[b])


    def wait_in(b):
      pltpu.make_async_copy(x_hbm.at[pl.ds(0, NB)], xbufs[b],
                            xin_sem.at[b]).wait()
      pltpu.make_async_copy(t_hbm.at[pl.ds(0, NB * N_PLAYERS)],
                            tbufs[b].at[pl.ds(0, NB * N_PLAYERS)],
                            tin_sem.at[b]).wait()

    def start_out(g, b):
      pltpu.async_copy(obufs[b],
                       out_hbm.at[pl.ds(base + g * NB, NB)],
                       out_sem.at[b])

    def wait_out(b):
      pltpu.make_async_copy(obufs[b], out_hbm.at[pl.ds(0, NB)],
                            out_sem.at[b]).wait()

    def compute(b):
      @plsc.parallel_loop(0, NB, unroll=2)
      def batch_body(bi):
        tv = tbufs[b][pl.ds(bi * N_PLAYERS, 16)]
        for p in range(N_PLAYERS):
          t = tv[p]
          cib = (p * N_TYPES + t) * EMBED
          xr = xbufs[b].at[bi, p]
          orr = obufs[b].at[bi, p]
          for jb in range(EMBED // 16):
            sl = pl.ds(jb * 16, 16)
            orr[sl] = xr[sl] + cbuf[pl.ds(cib + jb * 16, 16)]

    start_in(0, 0)
    start_in(1, 1)

    def pair_body(gg, _):
      for b in range(2):
        g = 2 * gg + b
        wait_in(b)

        @pl.when(gg >= 1)
        def _():
          wait_out(b)

        compute(b)
        start_out(g, b)

        @pl.when(g + 2 < NCH)
        def _():
          start_in(g + 2, b)
      return 0

    lax.fori_loop(0, NCH // 2, pair_body, 0)
    wait_out(0)
    wait_out(1)

  return _sc_add


def kernel(x, entity_types, entity_type_embedding, position_embedding):
  t_flat = entity_types.reshape(-1).astype(jnp.int32)
  out = _build_sc_add()(x, t_flat,
                        entity_type_embedding.reshape(-1),
                        position_embedding.reshape(-1))
  return out


# speedup vs baseline: 3.0752x; 3.0140x over previous
import functools

import jax
import jax.numpy as jnp
from jax import lax
from jax.experimental import pallas as pl
from jax.experimental.pallas import tpu as pltpu
from jax.experimental.pallas import tpu_sc as plsc

EMBED = 128
N_PLAYERS = 6
N_TYPES = 3
BATCH = 16384
NC, NS = 2, 16
NW = NC * NS
B_PER_W = BATCH // NW
NBB = 128
CPP = B_PER_W // NBB
NU = N_PLAYERS * CPP


@functools.cache
def _build_sc_add():
  mesh = plsc.VectorSubcoreMesh(core_axis_name="c", subcore_axis_name="s")

  @functools.partial(
      pl.kernel,
      out_type=jax.ShapeDtypeStruct((N_PLAYERS, BATCH, EMBED), jnp.float32),
      mesh=mesh,
      compiler_params=pltpu.CompilerParams(
          needs_layout_passes=False,
          skip_device_barrier=True,
          disable_bounds_checks=True,
      ),
      scratch_types=[
          pltpu.VMEM((NBB, EMBED), jnp.float32),
          pltpu.VMEM((NBB, EMBED), jnp.float32),
          pltpu.VMEM((NBB, EMBED), jnp.float32),
          pltpu.VMEM((NBB, EMBED), jnp.float32),
          pltpu.VMEM((NBB + 16,), jnp.int32),
          pltpu.VMEM((NBB + 16,), jnp.int32),
          pltpu.VMEM((N_PLAYERS * EMBED,), jnp.float32),
          pltpu.VMEM((N_TYPES * EMBED,), jnp.float32),
          pltpu.VMEM((N_PLAYERS * N_TYPES * EMBED,), jnp.float32),
          pltpu.SemaphoreType.DMA((2,)),
          pltpu.SemaphoreType.DMA((2,)),
          pltpu.SemaphoreType.DMA((2,)),
      ],
  )
  def _sc_add(x_hbm, t_hbm, te_hbm, pe_hbm, out_hbm,
              xbuf0, xbuf1, obuf0, obuf1, tbuf0, tbuf1,
              pe_v, te_v, cbuf,
              xin_sem, tin_sem, out_sem):
    xbufs = (xbuf0, xbuf1)
    obufs = (obuf0, obuf1)
    tbufs = (tbuf0, tbuf1)
    wid = lax.axis_index("s") * NC + lax.axis_index("c")
    base_b = wid * B_PER_W

    pltpu.sync_copy(pe_hbm, pe_v)
    pltpu.sync_copy(te_hbm, te_v)
    for p in range(N_PLAYERS):
      for t in range(N_TYPES):
        for j in range(EMBED // 16):
          cbuf[pl.ds((p * N_TYPES + t) * EMBED + j * 16, 16)] = (
              pe_v[pl.ds(p * EMBED + j * 16, 16)]
              + te_v[pl.ds(t * EMBED + j * 16, 16)])

    def start_in(u, b):
      p = u // CPP
      b0 = base_b + (u % CPP) * NBB
      pltpu.async_copy(x_hbm.at[p, pl.ds(b0, NBB)], xbufs[b], xin_sem.at[b])
      pltpu.async_copy(t_hbm.at[pl.ds(p * BATCH + b0, NBB)],
                       tbufs[b].at[pl.ds(0, NBB)], tin_sem.at[b])

    def wait_in(b):
      pltpu.make_async_copy(x_hbm.at[0, pl.ds(0, NBB)], xbufs[b],
                            xin_sem.at[b]).wait()
      pltpu.make_async_copy(t_hbm.at[pl.ds(0, NBB)],
                            tbufs[b].at[pl.ds(0, NBB)], tin_sem.at[b]).wait()

    def start_out(u, b):
      p = u // CPP
      b0 = base_b + (u % CPP) * NBB
      pltpu.async_copy(obufs[b], out_hbm.at[p, pl.ds(b0, NBB)],
                       out_sem.at[b])

    def wait_out(b):
      pltpu.make_async_copy(obufs[b], out_hbm.at[0, pl.ds(0, NBB)],
                            out_sem.at[b]).wait()

    def compute(u, b):
      p3 = (u // CPP) * N_TYPES

      @plsc.parallel_loop(0, NBB, unroll=2)
      def row_body(r):
        t = tbufs[b][pl.ds(r, 16)][0]
        cib = (p3 + t) * EMBED
        for jb in range(EMBED // 16):
          sl = pl.ds(jb * 16, 16)
          obufs[b][r, sl] = xbufs[b][r, sl] + cbuf[pl.ds(cib + jb * 16, 16)]

    start_in(0, 0)
    start_in(1, 1)

    def pair_body(uu, _):
      for b in range(2):
        u = 2 * uu + b
        wait_in(b)

        @pl.when(uu >= 1)
        def _():
          wait_out(b)

        compute(u, b)
        start_out(u, b)

        @pl.when(u + 2 < NU)
        def _():
          start_in(u + 2, b)
      return 0

    lax.fori_loop(0, NU // 2, pair_body, 0)
    wait_out(0)
    wait_out(1)

  return _sc_add


def kernel(x, entity_types, entity_type_embedding, position_embedding):
  xt = jnp.transpose(x, (1, 0, 2))
  tt = entity_types.T.reshape(-1).astype(jnp.int32)
  out = _build_sc_add()(xt, tt,
                        entity_type_embedding.reshape(-1),
                        position_embedding.reshape(-1))
  return jnp.transpose(out, (1, 0, 2))
